# baseline (speedup 1.0000x reference)
import functools

import jax
import jax.numpy as jnp
from jax import lax
from jax.experimental import pallas as pl
from jax.experimental.pallas import tpu as pltpu
from jax.experimental.pallas import tpu_sc as plsc

_HIDDEN = 64
_BATCH = 16384
_NC, _NS, _L = 2, 16, 16
_NW = _NC * _NS
_BPW = _BATCH // _NW
_NGROUP = _BPW // _L

_mesh = plsc.VectorSubcoreMesh(core_axis_name="c", subcore_axis_name="s")


@functools.partial(
    pl.kernel,
    mesh=_mesh,
    compiler_params=pltpu.CompilerParams(
        needs_layout_passes=False, use_tc_tiling_on_sc=True),
    out_type=jax.ShapeDtypeStruct((_BATCH,), jnp.float32),
    scratch_types=[
        pltpu.VMEM((_BPW,), jnp.int32),
        pltpu.VMEM((_BPW, _HIDDEN), jnp.float32),
        pltpu.VMEM((_HIDDEN,), jnp.float32),
        pltpu.VMEM((_L,), jnp.float32),
        pltpu.VMEM((_BPW,), jnp.float32),
        pltpu.SemaphoreType.DMA,
    ],
)
def _sc_lookup_project(ids_hbm, tab_hbm, w_hbm, b_hbm, out_hbm,
                       idx_v, rows_v, w_v, b_v, out_v, sem):
    wid = lax.axis_index("s") * _NC + lax.axis_index("c")

    pltpu.sync_copy(w_hbm, w_v)
    pltpu.sync_copy(b_hbm, b_v)
    pltpu.sync_copy(ids_hbm.at[pl.ds(wid * _BPW, _BPW)], idx_v)

    iota16 = lax.iota(jnp.int32, _L)
    bias_vec = b_v[...]
    w_chunks = [w_v[pl.ds(c * _L, _L)] for c in range(_HIDDEN // _L)]

    def fire(g):
        ids_vec = idx_v[pl.ds(g * _L, _L)]
        base = g * _L
        for j in range(_L):
            pltpu.async_copy(tab_hbm.at[pl.ds(ids_vec[j], 1)],
                             rows_v.at[pl.ds(base + j, 1)], sem)

    def drain(g):
        pltpu.make_async_copy(tab_hbm.at[pl.ds(0, _L)],
                              rows_v.at[pl.ds(g * _L, _L)], sem).wait()

    def compute(g):
        rows_idx = g * _L + iota16
        acc = bias_vec
        for h in range(_HIDDEN):
            col = plsc.load_gather(
                rows_v, [rows_idx, jnp.full((_L,), h, jnp.int32)])
            acc = acc + col * w_chunks[h // _L][h % _L]
        out_v[pl.ds(g * _L, _L)] = acc

    fire(0)

    def group_body(g, carry):
        fire(g + 1)
        drain(g)
        compute(g)
        return carry

    lax.fori_loop(0, _NGROUP - 1, group_body, 0)
    drain(_NGROUP - 1)
    compute(_NGROUP - 1)

    pltpu.sync_copy(out_v, out_hbm.at[pl.ds(wid * _BPW, _BPW)])


def kernel(item_ids, emb_table, fc_w, fc_b):
    ids = item_ids.astype(jnp.int32)
    w = fc_w.reshape(_HIDDEN).astype(jnp.float32)
    bias_vec = jnp.broadcast_to(fc_b.astype(jnp.float32), (_L,))
    out = _sc_lookup_project(ids, emb_table, w, bias_vec)
    return out.reshape(_BATCH, 1)


# speedup vs baseline: 2.6373x; 2.6373x over previous
import functools

import jax
import jax.numpy as jnp
from jax import lax
from jax.experimental import pallas as pl
from jax.experimental.pallas import tpu as pltpu
from jax.experimental.pallas import tpu_sc as plsc

_HIDDEN = 64
_BATCH = 16384
_NITEMS = 1000000
_NC, _NS, _L = 2, 16, 16
_NW = _NC * _NS
_BPW = _BATCH // _NW
_NCHUNK = 4
_CHUNK = _BPW // _NCHUNK

_BLK = 32768
_GRID = (_NITEMS + _BLK - 1) // _BLK


def _project_body(w_ref, tabT_ref, out_ref):
    out_ref[...] = jnp.dot(w_ref[...], tabT_ref[...],
                           preferred_element_type=jnp.float32)


_tc_project = pl.pallas_call(
    _project_body,
    grid=(_GRID,),
    in_specs=[
        pl.BlockSpec((1, _HIDDEN), lambda j: (0, 0)),
        pl.BlockSpec((_HIDDEN, _BLK), lambda j: (0, j)),
    ],
    out_specs=pl.BlockSpec((1, _BLK), lambda j: (0, j)),
    out_shape=jax.ShapeDtypeStruct((1, _NITEMS), jnp.float32),
)

_mesh = plsc.VectorSubcoreMesh(core_axis_name="c", subcore_axis_name="s")


@functools.partial(
    pl.kernel,
    mesh=_mesh,
    out_type=jax.ShapeDtypeStruct((_BATCH,), jnp.float32),
    scratch_types=[
        pltpu.VMEM((_NCHUNK, _CHUNK), jnp.int32),
        pltpu.VMEM((_BPW,), jnp.float32),
        pltpu.VMEM((_L,), jnp.float32),
        pltpu.SemaphoreType.DMA,
    ],
)
def _sc_gather_bias(ids_hbm, proj_hbm, b_hbm, out_hbm,
                    idx_v, vals_v, b_v, sem):
    wid = lax.axis_index("s") * _NC + lax.axis_index("c")

    pltpu.sync_copy(b_hbm, b_v)
    pltpu.sync_copy(ids_hbm.at[pl.ds(wid * _NCHUNK, _NCHUNK)], idx_v)

    copies = [
        pltpu.async_copy(proj_hbm.at[idx_v.at[j]],
                         vals_v.at[pl.ds(j * _CHUNK, _CHUNK)], sem)
        for j in range(_NCHUNK)
    ]
    for c in copies:
        c.wait()

    bias_vec = b_v[...]
    for v in range(_BPW // _L):
        vals_v[pl.ds(v * _L, _L)] = vals_v[pl.ds(v * _L, _L)] + bias_vec

    pltpu.sync_copy(vals_v, out_hbm.at[pl.ds(wid * _BPW, _BPW)])


def kernel(item_ids, emb_table, fc_w, fc_b):
    ids2d = item_ids.astype(jnp.int32).reshape(_NW * _NCHUNK, _CHUNK)
    tabT = emb_table.T
    proj = _tc_project(fc_w.astype(jnp.float32), tabT).reshape(_NITEMS)
    bias_vec = jnp.broadcast_to(fc_b.astype(jnp.float32), (_L,))
    out = _sc_gather_bias(ids2d, proj, bias_vec)
    return out.reshape(_BATCH, 1)


# speedup vs baseline: 3.8404x; 1.4562x over previous
import functools

import jax
import jax.numpy as jnp
from jax import lax
from jax.experimental import pallas as pl
from jax.experimental.pallas import tpu as pltpu
from jax.experimental.pallas import tpu_sc as plsc

_HIDDEN = 64
_BATCH = 16384
_NITEMS = 1000000
_NC, _NS, _L = 2, 16, 16
_NW = _NC * _NS
_BPW = _BATCH // _NW
_NCHUNK = 4
_CHUNK = _BPW // _NCHUNK

_BLK = 32768
_GRID = (_NITEMS + _BLK - 1) // _BLK


def _project_body(w_ref, tabT_ref, out_ref):
    out_ref[...] = jnp.dot(w_ref[...], tabT_ref[...],
                           preferred_element_type=jnp.float32)[0]


_tc_project = pl.pallas_call(
    _project_body,
    grid=(_GRID,),
    in_specs=[
        pl.BlockSpec((1, _HIDDEN), lambda j: (0, 0)),
        pl.BlockSpec((_HIDDEN, _BLK), lambda j: (0, j)),
    ],
    out_specs=pl.BlockSpec((_BLK,), lambda j: (j,)),
    out_shape=jax.ShapeDtypeStruct((_NITEMS,), jnp.float32),
)

_mesh = plsc.VectorSubcoreMesh(core_axis_name="c", subcore_axis_name="s")


@functools.partial(
    pl.kernel,
    mesh=_mesh,
    out_type=jax.ShapeDtypeStruct((_BATCH,), jnp.float32),
    scratch_types=[
        pltpu.VMEM((_NCHUNK, _CHUNK), jnp.int32),
        pltpu.VMEM((_BPW,), jnp.float32),
        pltpu.VMEM((_L,), jnp.float32),
        pltpu.SemaphoreType.DMA,
    ],
)
def _sc_gather_bias(ids_hbm, proj_hbm, b_hbm, out_hbm,
                    idx_v, vals_v, b_v, sem):
    wid = lax.axis_index("s") * _NC + lax.axis_index("c")

    pltpu.sync_copy(b_hbm, b_v)
    pltpu.sync_copy(ids_hbm.at[pl.ds(wid * _NCHUNK, _NCHUNK)], idx_v)

    copies = [
        pltpu.async_copy(proj_hbm.at[idx_v.at[j]],
                         vals_v.at[pl.ds(j * _CHUNK, _CHUNK)], sem)
        for j in range(_NCHUNK)
    ]
    for c in copies:
        c.wait()

    bias_vec = b_v[...]
    for v in range(_BPW // _L):
        vals_v[pl.ds(v * _L, _L)] = vals_v[pl.ds(v * _L, _L)] + bias_vec

    pltpu.sync_copy(vals_v, out_hbm.at[pl.ds(wid * _BPW, _BPW)])


def kernel(item_ids, emb_table, fc_w, fc_b):
    ids2d = item_ids.astype(jnp.int32).reshape(_NW * _NCHUNK, _CHUNK)
    tabT = emb_table.T
    proj = _tc_project(fc_w.astype(jnp.float32), tabT)
    bias_vec = jnp.broadcast_to(fc_b.astype(jnp.float32), (_L,))
    out = _sc_gather_bias(ids2d, proj, bias_vec)
    return out.reshape(_BATCH, 1)
